# baseline (speedup 1.0000x reference)
import functools
import itertools

import numpy as np
import jax
import jax.numpy as jnp
from jax import lax
from jax.experimental import pallas as pl
from jax.experimental.pallas import tpu as pltpu
from jax.experimental.pallas import tpu_sc as plsc

_B, _T, _D, _F, _E, _K = 2, 2048, 768, 3072, 8, 2
_N = _B * _T
_TB = 512
_NC = 32
_CT = _N // _NC
_BLK = 512
_NBMAX = _N * _K // _BLK + _E
_AMAX = _NBMAX * _BLK
_NB_PAD = 48
_META = 128
_FSPLIT = 1

_SC_PARAMS = pltpu.CompilerParams(needs_layout_passes=False)


def _trig_np():
    signs = [-1.0, 1.0]
    t = np.array(list(itertools.product(signs, repeat=3)), dtype=np.float32)
    t = t / np.linalg.norm(t, axis=1, keepdims=True)
    return t[:_E]


def _router_body(x_ref, wrt_ref, trg_ref, tki_ref, tkw_ref, psum_ref):
    c = pl.program_id(0)
    xb = x_ref[...]
    z3 = lax.dot_general(xb, wrt_ref[...], (((1,), (0,)), ((), ())),
                         precision=lax.Precision.DEFAULT,
                         preferred_element_type=jnp.float32)
    lgT = lax.dot_general(trg_ref[...], z3, (((1,), (1,)), ((), ())),
                          precision=lax.Precision.DEFAULT,
                          preferred_element_type=jnp.float32)
    io8 = lax.broadcasted_iota(jnp.int32, (_E, _TB), 0)
    m1 = jnp.max(lgT, axis=0, keepdims=True)
    a1 = jnp.min(jnp.where(lgT == m1, io8, _E), axis=0, keepdims=True)
    lg2 = jnp.where(io8 == a1, jnp.float32(-1e30), lgT)
    m2 = jnp.max(lg2, axis=0, keepdims=True)
    a2 = jnp.min(jnp.where(lg2 == m2, io8, _E), axis=0, keepdims=True)
    w1 = 1.0 / (1.0 + jnp.exp(m2 - m1))
    tki_ref[0:1, :] = a1
    tki_ref[1:2, :] = a2
    tkw_ref[0:1, :] = w1
    tkw_ref[1:2, :] = 1.0 - w1
    el = jnp.exp(lgT - m1)
    probs = el / jnp.sum(el, axis=0, keepdims=True)
    ones = jnp.ones((_TB, 128), jnp.float32)
    ps = lax.dot_general(probs, ones, (((1,), (0,)), ((), ())),
                         precision=lax.Precision.HIGHEST,
                         preferred_element_type=jnp.float32)

    @pl.when(c == 0)
    def _():
        psum_ref[...] = jnp.zeros_like(psum_ref)

    psum_ref[...] += ps


def _router(xf, wrt, trg, interpret=False):
    return pl.pallas_call(
        _router_body,
        grid=(_N // _TB,),
        in_specs=[
            pl.BlockSpec((_TB, _D), lambda c: (c, 0)),
            pl.BlockSpec((_D, _E), lambda c: (0, 0)),
            pl.BlockSpec((_E, _E), lambda c: (0, 0)),
        ],
        out_specs=[
            pl.BlockSpec((_K, _TB), lambda c: (0, c)),
            pl.BlockSpec((_K, _TB), lambda c: (0, c)),
            pl.BlockSpec((_E, 128), lambda c: (0, 0)),
        ],
        out_shape=[
            jax.ShapeDtypeStruct((_K, _N), jnp.int32),
            jax.ShapeDtypeStruct((_K, _N), jnp.float32),
            jax.ShapeDtypeStruct((_E, 128), jnp.float32),
        ],
        interpret=interpret,
    )(xf, wrt, trg)


def _dispatch(tki, xf):
    mesh = plsc.VectorSubcoreMesh(core_axis_name="c", subcore_axis_name="s")

    @functools.partial(
        pl.kernel,
        out_type=[
            jax.ShapeDtypeStruct((_AMAX, _D), jnp.float32),
            jax.ShapeDtypeStruct((_K, _N), jnp.int32),
            jax.ShapeDtypeStruct((_META,), jnp.int32),
        ],
        mesh=mesh,
        scratch_types=[
            pltpu.VMEM((_K * _N,), jnp.int32),
            pltpu.VMEM((4, 16), jnp.int32),
            pltpu.VMEM((4, 16), jnp.int32),
            pltpu.VMEM((_K, _CT), jnp.int32),
            pltpu.VMEM((4, 64), jnp.int32),
            pltpu.VMEM((64, _D), jnp.float32),
            pltpu.VMEM((64, _D), jnp.float32),
            pltpu.VMEM((_META,), jnp.int32),
            pltpu.SemaphoreType.DMA,
            pltpu.SemaphoreType.DMA,
            pltpu.SemaphoreType.DMA,
        ],
        compiler_params=_SC_PARAMS,
    )
    def k(tki_hbm, x_hbm, xs_hbm, dest_hbm, meta_hbm,
          ids_v, sv_v, hst_v, dst_v, idx_v, xb_v, xb2_v, meta_v,
          sem0, sem1, sem2):
        cid = lax.axis_index("c")
        sid = lax.axis_index("s")
        c = sid * 2 + cid
        bufs = [xb_v, xb2_v]
        lds = [pltpu.async_copy(x_hbm.at[pl.ds(c * _CT + s * 64, 64)],
                                bufs[s], sem2) for s in range(2)]
        for kk in range(_K):
            pltpu.sync_copy(tki_hbm.at[kk], ids_v.at[pl.ds(kk * _N, _N)])

        lane = lax.iota(jnp.int32, 16)
        zero16 = jnp.zeros((16,), jnp.int32)

        for r in range(3):
            hst_v[r, :] = zero16
        one16 = jnp.ones((16,), jnp.int32)

        def scan_body(i, _):
            vec = ids_v[pl.ds(i * 16, 16)]
            m0 = jnp.full((16,), i < 8 * c, jnp.bool_)
            m1b = jnp.full((16,), i < 256 + 8 * c, jnp.bool_)
            plsc.addupdate_scatter(hst_v.at[0], [vec], one16)
            plsc.addupdate_scatter(hst_v.at[1], [vec], one16, mask=m0)
            plsc.addupdate_scatter(hst_v.at[2], [vec], one16, mask=m1b)
            return 0

        lax.fori_loop(0, (_K * _N) // 16, scan_body, 0)
        tot = hst_v[0, :]
        p0 = hst_v[1, :]
        p1 = hst_v[2, :]

        pc = jnp.bitwise_and(tot + (_BLK - 1), jnp.int32(-_BLK))
        cum = plsc.cumsum(pc)
        pad_off = cum - pc
        sv_v[0, :] = pad_off + p0
        sv_v[1, :] = pad_off + p1
        sv_v[2, :] = cum

        for kk in range(_K):
            srow = sv_v[kk, :]
            starts = [jnp.full((16,), srow[e], jnp.int32) for e in range(_E)]
            for j in range(_CT // 16):
                vec = ids_v[pl.ds(kk * _N + c * _CT + j * 16, 16)]
                dst = zero16
                for e in range(_E):
                    m = vec == e
                    r = plsc.cumsum(jnp.where(m, 1, 0))
                    dst = dst + jnp.where(m, starts[e] + r - 1, zero16)
                    starts[e] = starts[e] + plsc.all_reduce_population_count(m)
                dst_v[kk, pl.ds(j * 16, 16)] = dst
                idx_v[kk * 2 + j // 4, pl.ds((j % 4) * 16, 16)] = dst
            pltpu.sync_copy(dst_v.at[kk], dest_hbm.at[kk, pl.ds(c * _CT, _CT)])

        @pl.when(c == 0)
        def _():
            cumv = sv_v[2, :]
            pt = cumv[_E - 1]
            nb = lax.shift_right_logical(pt, _BLK.bit_length() - 1)
            s_last = (nb - 1) * _BLK
            eb_last = jnp.int32(0)
            for e in range(_E):
                eb_last = eb_last + jnp.where(cumv[e] <= s_last, 1, 0)
            for j in range(3):
                bi = lane + 16 * j
                s = bi * _BLK
                eb = zero16
                for e in range(_E):
                    eb = eb + jnp.where(jnp.full((16,), cumv[e], jnp.int32) <= s, 1, 0)
                meta_v[pl.ds(16 * j, 16)] = jnp.minimum(eb, jnp.full((16,), eb_last, jnp.int32))
                meta_v[pl.ds(_NB_PAD + 16 * j, 16)] = jnp.minimum(bi, jnp.full((16,), nb - 1, jnp.int32))
            meta_v[pl.ds(96, 16)] = jnp.where(lane == 0, jnp.full((16,), nb, jnp.int32), zero16)
            meta_v[pl.ds(112, 16)] = zero16
            pltpu.sync_copy(meta_v, meta_hbm)

        cps = []
        for sub in range(2):
            lds[sub].wait()
            cps.append(pltpu.async_copy(bufs[sub], xs_hbm.at[idx_v.at[sub]], sem0))
            cps.append(pltpu.async_copy(bufs[sub], xs_hbm.at[idx_v.at[2 + sub]], sem1))
        for cp in cps:
            cp.wait()

    return k(tki, xf)


def _ffn_body(meta_ref, xs_ref, w1_ref, w2_ref, ys_ref):
    b = pl.program_id(0)
    nb = meta_ref[96]

    @pl.when(b < nb)
    def _():
        xg = xs_ref[...]
        fs = _F // _FSPLIT
        acc = jnp.zeros((_BLK, _D), jnp.float32)
        for fi in range(_FSPLIT):
            w1s = w1_ref[0, pl.ds(fi * fs, fs), :]
            h = lax.dot_general(xg, w1s, (((1,), (1,)), ((), ())),
                                precision=lax.Precision.DEFAULT,
                                preferred_element_type=jnp.float32)
            h = 0.5 * h * (1.0 + lax.erf(h * np.float32(1.0 / np.sqrt(2.0))))
            w2s = w2_ref[0, :, pl.ds(fi * fs, fs)]
            acc = acc + lax.dot_general(h, w2s, (((1,), (1,)), ((), ())),
                                        precision=lax.Precision.DEFAULT,
                                        preferred_element_type=jnp.float32)
        ys_ref[...] = acc


def _ffn(meta, xs, W1, W2, interpret=False):
    grid_spec = pltpu.PrefetchScalarGridSpec(
        num_scalar_prefetch=1,
        grid=(_NBMAX,),
        in_specs=[
            pl.BlockSpec((_BLK, _D), lambda b, m: (m[_NB_PAD + b], 0)),
            pl.BlockSpec((1, _F, _D), lambda b, m: (m[b], 0, 0)),
            pl.BlockSpec((1, _D, _F), lambda b, m: (m[b], 0, 0)),
        ],
        out_specs=pl.BlockSpec((_BLK, _D), lambda b, m: (b, 0)),
    )
    return pl.pallas_call(
        _ffn_body,
        grid_spec=grid_spec,
        out_shape=jax.ShapeDtypeStruct((_AMAX, _D), jnp.float32),
        interpret=interpret,
    )(meta, xs, W1, W2)


def _combine(ys, dest, tkw):
    mesh = plsc.VectorSubcoreMesh(core_axis_name="c", subcore_axis_name="s")

    @functools.partial(
        pl.kernel,
        out_type=jax.ShapeDtypeStruct((_N, _D), jnp.float32),
        mesh=mesh,
        scratch_types=[
            pltpu.VMEM((_K, _CT), jnp.int32),
            pltpu.VMEM((_K, _CT), jnp.float32),
            pltpu.VMEM((16, _D), jnp.float32),
            pltpu.VMEM((16, _D), jnp.float32),
            pltpu.VMEM((16, _D), jnp.float32),
            pltpu.VMEM((16, _D), jnp.float32),
            pltpu.VMEM((16, _D), jnp.float32),
            pltpu.VMEM((16, _D), jnp.float32),
            pltpu.VMEM((16, _D), jnp.float32),
            pltpu.VMEM((16, _D), jnp.float32),
            pltpu.SemaphoreType.DMA,
            pltpu.SemaphoreType.DMA,
            pltpu.SemaphoreType.DMA,
            pltpu.SemaphoreType.DMA,
            pltpu.SemaphoreType.DMA,
        ],
        compiler_params=_SC_PARAMS,
    )
    def k(ys_hbm, dest_hbm, tkw_hbm, out_hbm,
          di_v, w_v, r00, r01, r02, r10, r11, r12, ob0, ob1,
          gs0, gs1, gs2, os0, os1):
        cid = lax.axis_index("c")
        sid = lax.axis_index("s")
        c = sid * 2 + cid
        base = c * _CT
        for kk in range(_K):
            pltpu.sync_copy(dest_hbm.at[kk, pl.ds(base, _CT)], di_v.at[kk])
            pltpu.sync_copy(tkw_hbm.at[kk, pl.ds(base, _CT)], w_v.at[kk])

        rbuf = [[r00, r01, r02], [r10, r11, r12]]
        obuf = [ob0, ob1]
        gsem = [gs0, gs1, gs2]
        osem = [os0, os1]
        nsub = _CT // 16
        ndeep = 3

        def fire(sub):
            pb = sub % ndeep
            g0 = pltpu.async_copy(ys_hbm.at[di_v.at[0, pl.ds(sub * 16, 16)]],
                                  rbuf[0][pb], gsem[pb])
            g1 = pltpu.async_copy(ys_hbm.at[di_v.at[1, pl.ds(sub * 16, 16)]],
                                  rbuf[1][pb], gsem[pb])
            return g0, g1

        cps = {0: fire(0), 1: fire(1)}
        ocps = {}
        for sub in range(nsub):
            pb = sub % ndeep
            if sub + 2 < nsub:
                cps[sub + 2] = fire(sub + 2)
            g0, g1 = cps[sub]
            g0.wait()
            g1.wait()
            if sub >= 2:
                ocps[sub - 2].wait()
            w0v = w_v[0, pl.ds(sub * 16, 16)]
            w1v = w_v[1, pl.ds(sub * 16, 16)]
            ob_i = sub % 2
            r0, r1, ob = rbuf[0][pb], rbuf[1][pb], obuf[ob_i]
            for i in range(16):
                w0 = w0v[i]
                w1 = w1v[i]

                @pl.loop(0, _D, step=16)
                def _(d0, i=i, w0=w0, w1=w1, r0=r0, r1=r1, ob=ob):
                    ob[i, pl.ds(d0, 16)] = (w0 * r0[i, pl.ds(d0, 16)]
                                            + w1 * r1[i, pl.ds(d0, 16)])

            ocps[sub] = pltpu.async_copy(ob, out_hbm.at[pl.ds(base + sub * 16, 16)],
                                         osem[ob_i])
        ocps[nsub - 2].wait()
        ocps[nsub - 1].wait()

    return k(ys, dest, tkw)


def kernel(x, Wr, W1, W2):
    xf = x.reshape(_N, _D)
    wrt = jnp.concatenate([Wr.T, jnp.zeros((_D, _E - 3), jnp.float32)], axis=1)
    trg = jnp.asarray(np.pad(_trig_np(), ((0, 0), (0, _E - 3))))
    tki, tkw, psum = _router(xf, wrt, trg)
    xs, dest, meta = _dispatch(tki, xf)
    ys = _ffn(meta, xs, W1, W2)
    out = _combine(ys, dest, tkw)
    tpe = psum[:, 0] / jnp.float32(_N)
    aux = 0.01 * jnp.mean((tpe - jnp.float32(1.0 / _E)) ** 2)
    return out.reshape(_B, _T, _D), aux


# speedup vs baseline: 1.0031x; 1.0031x over previous
import functools
import itertools

import numpy as np
import jax
import jax.numpy as jnp
from jax import lax
from jax.experimental import pallas as pl
from jax.experimental.pallas import tpu as pltpu
from jax.experimental.pallas import tpu_sc as plsc

_B, _T, _D, _F, _E, _K = 2, 2048, 768, 3072, 8, 2
_N = _B * _T
_TB = 512
_NC = 32
_CT = _N // _NC
_BLK = 512
_NBMAX = _N * _K // _BLK + _E
_AMAX = _NBMAX * _BLK
_NB_PAD = 48
_META = 128
_FSPLIT = 1

_SC_PARAMS = pltpu.CompilerParams(needs_layout_passes=False)


def _trig_np():
    signs = [-1.0, 1.0]
    t = np.array(list(itertools.product(signs, repeat=3)), dtype=np.float32)
    t = t / np.linalg.norm(t, axis=1, keepdims=True)
    return t[:_E]


def _router_body(x_ref, wrt_ref, trg_ref, tki_ref, tkw_ref, psum_ref):
    c = pl.program_id(0)
    xb = x_ref[...]
    z3 = lax.dot_general(xb, wrt_ref[...], (((1,), (0,)), ((), ())),
                         precision=lax.Precision.DEFAULT,
                         preferred_element_type=jnp.float32)
    lgT = lax.dot_general(trg_ref[...], z3, (((1,), (1,)), ((), ())),
                          precision=lax.Precision.DEFAULT,
                          preferred_element_type=jnp.float32)
    io8 = lax.broadcasted_iota(jnp.int32, (_E, _TB), 0)
    m1 = jnp.max(lgT, axis=0, keepdims=True)
    a1 = jnp.min(jnp.where(lgT == m1, io8, _E), axis=0, keepdims=True)
    lg2 = jnp.where(io8 == a1, jnp.float32(-1e30), lgT)
    m2 = jnp.max(lg2, axis=0, keepdims=True)
    a2 = jnp.min(jnp.where(lg2 == m2, io8, _E), axis=0, keepdims=True)
    w1 = 1.0 / (1.0 + jnp.exp(m2 - m1))
    tki_ref[0:1, :] = a1
    tki_ref[1:2, :] = a2
    tkw_ref[0:1, :] = w1
    tkw_ref[1:2, :] = 1.0 - w1
    el = jnp.exp(lgT - m1)
    probs = el / jnp.sum(el, axis=0, keepdims=True)
    ones = jnp.ones((_TB, 128), jnp.float32)
    ps = lax.dot_general(probs, ones, (((1,), (0,)), ((), ())),
                         precision=lax.Precision.HIGHEST,
                         preferred_element_type=jnp.float32)

    @pl.when(c == 0)
    def _():
        psum_ref[...] = jnp.zeros_like(psum_ref)

    psum_ref[...] += ps


def _router(xf, wrt, trg, interpret=False):
    return pl.pallas_call(
        _router_body,
        grid=(_N // _TB,),
        in_specs=[
            pl.BlockSpec((_TB, _D), lambda c: (c, 0)),
            pl.BlockSpec((_D, _E), lambda c: (0, 0)),
            pl.BlockSpec((_E, _E), lambda c: (0, 0)),
        ],
        out_specs=[
            pl.BlockSpec((_K, _TB), lambda c: (0, c)),
            pl.BlockSpec((_K, _TB), lambda c: (0, c)),
            pl.BlockSpec((_E, 128), lambda c: (0, 0)),
        ],
        out_shape=[
            jax.ShapeDtypeStruct((_K, _N), jnp.int32),
            jax.ShapeDtypeStruct((_K, _N), jnp.float32),
            jax.ShapeDtypeStruct((_E, 128), jnp.float32),
        ],
        interpret=interpret,
    )(xf, wrt, trg)


def _dispatch(tki, xf):
    mesh = plsc.VectorSubcoreMesh(core_axis_name="c", subcore_axis_name="s")

    @functools.partial(
        pl.kernel,
        out_type=[
            jax.ShapeDtypeStruct((_AMAX, _D), jnp.float32),
            jax.ShapeDtypeStruct((_K, _N), jnp.int32),
            jax.ShapeDtypeStruct((_META,), jnp.int32),
        ],
        mesh=mesh,
        scratch_types=[
            pltpu.VMEM((_K * _N,), jnp.int32),
            pltpu.VMEM((4, 16), jnp.int32),
            pltpu.VMEM((4, 16), jnp.int32),
            pltpu.VMEM((_K, _CT), jnp.int32),
            pltpu.VMEM((4, 64), jnp.int32),
            pltpu.VMEM((64, _D), jnp.float32),
            pltpu.VMEM((64, _D), jnp.float32),
            pltpu.VMEM((_META,), jnp.int32),
            pltpu.SemaphoreType.DMA,
            pltpu.SemaphoreType.DMA,
            pltpu.SemaphoreType.DMA,
        ],
        compiler_params=_SC_PARAMS,
    )
    def k(tki_hbm, x_hbm, xs_hbm, dest_hbm, meta_hbm,
          ids_v, sv_v, hst_v, dst_v, idx_v, xb_v, xb2_v, meta_v,
          sem0, sem1, sem2):
        cid = lax.axis_index("c")
        sid = lax.axis_index("s")
        c = sid * 2 + cid
        bufs = [xb_v, xb2_v]
        lds = [pltpu.async_copy(x_hbm.at[pl.ds(c * _CT + s * 64, 64)],
                                bufs[s], sem2) for s in range(2)]
        for kk in range(_K):
            pltpu.sync_copy(tki_hbm.at[kk], ids_v.at[pl.ds(kk * _N, _N)])

        lane = lax.iota(jnp.int32, 16)
        zero16 = jnp.zeros((16,), jnp.int32)

        for r in range(3):
            hst_v[r, :] = zero16
        one16 = jnp.ones((16,), jnp.int32)

        def scan_body(i, _):
            vec = ids_v[pl.ds(i * 16, 16)]
            m0 = jnp.full((16,), i < 8 * c, jnp.bool_)
            m1b = jnp.full((16,), i < 256 + 8 * c, jnp.bool_)
            plsc.addupdate_scatter(hst_v.at[0], [vec], one16)
            plsc.addupdate_scatter(hst_v.at[1], [vec], one16, mask=m0)
            plsc.addupdate_scatter(hst_v.at[2], [vec], one16, mask=m1b)
            return 0

        lax.fori_loop(0, (_K * _N) // 16, scan_body, 0)
        tot = hst_v[0, :]
        p0 = hst_v[1, :]
        p1 = hst_v[2, :]

        pc = jnp.bitwise_and(tot + (_BLK - 1), jnp.int32(-_BLK))
        cum = plsc.cumsum(pc)
        pad_off = cum - pc
        sv_v[0, :] = pad_off + p0
        sv_v[1, :] = pad_off + p1
        sv_v[2, :] = cum

        for kk in range(_K):
            srow = sv_v[kk, :]
            starts = [jnp.full((16,), srow[e], jnp.int32) for e in range(_E)]
            for j in range(_CT // 16):
                vec = ids_v[pl.ds(kk * _N + c * _CT + j * 16, 16)]
                dst = zero16
                for e in range(_E):
                    m = vec == e
                    r = plsc.cumsum(jnp.where(m, 1, 0))
                    dst = dst + jnp.where(m, starts[e] + r - 1, zero16)
                    starts[e] = starts[e] + plsc.all_reduce_population_count(m)
                dst_v[kk, pl.ds(j * 16, 16)] = dst
                idx_v[kk * 2 + j // 4, pl.ds((j % 4) * 16, 16)] = dst
            pltpu.sync_copy(dst_v.at[kk], dest_hbm.at[kk, pl.ds(c * _CT, _CT)])

        @pl.when(c == 0)
        def _():
            cumv = sv_v[2, :]
            pt = cumv[_E - 1]
            nb = lax.shift_right_logical(pt, _BLK.bit_length() - 1)
            s_last = (nb - 1) * _BLK
            eb_last = jnp.int32(0)
            for e in range(_E):
                eb_last = eb_last + jnp.where(cumv[e] <= s_last, 1, 0)
            for j in range(3):
                bi = lane + 16 * j
                s = bi * _BLK
                eb = zero16
                for e in range(_E):
                    eb = eb + jnp.where(jnp.full((16,), cumv[e], jnp.int32) <= s, 1, 0)
                meta_v[pl.ds(16 * j, 16)] = jnp.minimum(eb, jnp.full((16,), eb_last, jnp.int32))
                meta_v[pl.ds(_NB_PAD + 16 * j, 16)] = jnp.minimum(bi, jnp.full((16,), nb - 1, jnp.int32))
            meta_v[pl.ds(96, 16)] = jnp.where(lane == 0, jnp.full((16,), nb, jnp.int32), zero16)
            meta_v[pl.ds(112, 16)] = zero16
            pltpu.sync_copy(meta_v, meta_hbm)

        cps = []
        for sub in range(2):
            lds[sub].wait()
            cps.append(pltpu.async_copy(bufs[sub], xs_hbm.at[idx_v.at[sub]], sem0))
            cps.append(pltpu.async_copy(bufs[sub], xs_hbm.at[idx_v.at[2 + sub]], sem1))
        for cp in cps:
            cp.wait()

    return k(tki, xf)


def _ffn_body(meta_ref, xs_ref, w1_ref, w2_ref, ys_ref):
    b = pl.program_id(0)
    nb = meta_ref[96]

    @pl.when(b < nb)
    def _():
        xg = xs_ref[...]
        fs = _F // _FSPLIT
        acc = jnp.zeros((_BLK, _D), jnp.float32)
        for fi in range(_FSPLIT):
            w1s = w1_ref[0, pl.ds(fi * fs, fs), :]
            h = lax.dot_general(xg, w1s, (((1,), (1,)), ((), ())),
                                precision=lax.Precision.DEFAULT,
                                preferred_element_type=jnp.float32)
            h = 0.5 * h * (1.0 + lax.erf(h * np.float32(1.0 / np.sqrt(2.0))))
            w2s = w2_ref[0, :, pl.ds(fi * fs, fs)]
            acc = acc + lax.dot_general(h, w2s, (((1,), (1,)), ((), ())),
                                        precision=lax.Precision.DEFAULT,
                                        preferred_element_type=jnp.float32)
        ys_ref[...] = acc


def _ffn(meta, xs, W1, W2, interpret=False):
    grid_spec = pltpu.PrefetchScalarGridSpec(
        num_scalar_prefetch=1,
        grid=(_NBMAX,),
        in_specs=[
            pl.BlockSpec((_BLK, _D), lambda b, m: (m[_NB_PAD + b], 0)),
            pl.BlockSpec((1, _F, _D), lambda b, m: (m[b], 0, 0)),
            pl.BlockSpec((1, _D, _F), lambda b, m: (m[b], 0, 0)),
        ],
        out_specs=pl.BlockSpec((_BLK, _D), lambda b, m: (b, 0)),
    )
    return pl.pallas_call(
        _ffn_body,
        grid_spec=grid_spec,
        out_shape=jax.ShapeDtypeStruct((_AMAX, _D), jnp.float32),
        interpret=interpret,
    )(meta, xs, W1, W2)


def _combine(ys, dest, tkw):
    mesh = plsc.VectorSubcoreMesh(core_axis_name="c", subcore_axis_name="s")

    @functools.partial(
        pl.kernel,
        out_type=jax.ShapeDtypeStruct((_N, _D), jnp.float32),
        mesh=mesh,
        scratch_types=[
            pltpu.VMEM((_K, _CT), jnp.int32),
            pltpu.VMEM((_K, _CT), jnp.float32),
            pltpu.VMEM((16, _D), jnp.float32),
            pltpu.VMEM((16, _D), jnp.float32),
            pltpu.VMEM((16, _D), jnp.float32),
            pltpu.VMEM((16, _D), jnp.float32),
            pltpu.VMEM((16, _D), jnp.float32),
            pltpu.VMEM((16, _D), jnp.float32),
            pltpu.SemaphoreType.DMA,
            pltpu.SemaphoreType.DMA,
            pltpu.SemaphoreType.DMA,
            pltpu.SemaphoreType.DMA,
        ],
        compiler_params=_SC_PARAMS,
    )
    def k(ys_hbm, dest_hbm, tkw_hbm, out_hbm,
          di_v, w_v, r00, r01, r10, r11, ob0, ob1,
          gs0, gs1, os0, os1):
        cid = lax.axis_index("c")
        sid = lax.axis_index("s")
        c = sid * 2 + cid
        base = c * _CT
        for kk in range(_K):
            pltpu.sync_copy(dest_hbm.at[kk, pl.ds(base, _CT)], di_v.at[kk])
            pltpu.sync_copy(tkw_hbm.at[kk, pl.ds(base, _CT)], w_v.at[kk])

        rbuf = [[r00, r01], [r10, r11]]
        obuf = [ob0, ob1]
        gsem = [gs0, gs1]
        osem = [os0, os1]
        nsub = _CT // 16
        ndeep = 2

        def fire(sub):
            pb = sub % ndeep
            g0 = pltpu.async_copy(ys_hbm.at[di_v.at[0, pl.ds(sub * 16, 16)]],
                                  rbuf[0][pb], gsem[pb])
            g1 = pltpu.async_copy(ys_hbm.at[di_v.at[1, pl.ds(sub * 16, 16)]],
                                  rbuf[1][pb], gsem[pb])
            return g0, g1

        cps = {0: fire(0)}
        ocps = {}
        for sub in range(nsub):
            pb = sub % ndeep
            if sub + 1 < nsub:
                cps[sub + 1] = fire(sub + 1)
            g0, g1 = cps[sub]
            g0.wait()
            g1.wait()
            if sub >= 2:
                ocps[sub - 2].wait()
            w0v = w_v[0, pl.ds(sub * 16, 16)]
            w1v = w_v[1, pl.ds(sub * 16, 16)]
            ob_i = sub % 2
            r0, r1, ob = rbuf[0][pb], rbuf[1][pb], obuf[ob_i]
            for i in range(16):
                w0 = w0v[i]
                w1 = w1v[i]

                @pl.loop(0, _D, step=16)
                def _(d0, i=i, w0=w0, w1=w1, r0=r0, r1=r1, ob=ob):
                    ob[i, pl.ds(d0, 16)] = (w0 * r0[i, pl.ds(d0, 16)]
                                            + w1 * r1[i, pl.ds(d0, 16)])

            ocps[sub] = pltpu.async_copy(ob, out_hbm.at[pl.ds(base + sub * 16, 16)],
                                         osem[ob_i])
        ocps[nsub - 2].wait()
        ocps[nsub - 1].wait()

    return k(ys, dest, tkw)


def kernel(x, Wr, W1, W2):
    xf = x.reshape(_N, _D)
    wrt = jnp.concatenate([Wr.T, jnp.zeros((_D, _E - 3), jnp.float32)], axis=1)
    trg = jnp.asarray(np.pad(_trig_np(), ((0, 0), (0, _E - 3))))
    tki, tkw, psum = _router(xf, wrt, trg)
    xs, dest, meta = _dispatch(tki, xf)
    ys = _ffn(meta, xs, W1, W2)
    out = _combine(ys, dest, tkw)
    tpe = psum[:, 0] / jnp.float32(_N)
    aux = 0.01 * jnp.mean((tpe - jnp.float32(1.0 / _E)) ** 2)
    return out.reshape(_B, _T, _D), aux


# speedup vs baseline: 1.0065x; 1.0034x over previous
import functools
import itertools

import numpy as np
import jax
import jax.numpy as jnp
from jax import lax
from jax.experimental import pallas as pl
from jax.experimental.pallas import tpu as pltpu
from jax.experimental.pallas import tpu_sc as plsc

_B, _T, _D, _F, _E, _K = 2, 2048, 768, 3072, 8, 2
_N = _B * _T
_TB = 512
_NC = 32
_CT = _N // _NC
_BLK = 512
_NBMAX = _N * _K // _BLK + _E
_AMAX = _NBMAX * _BLK
_NB_PAD = 48
_META = 128
_FSPLIT = 1

_SC_PARAMS = pltpu.CompilerParams(needs_layout_passes=False)


def _trig_np():
    signs = [-1.0, 1.0]
    t = np.array(list(itertools.product(signs, repeat=3)), dtype=np.float32)
    t = t / np.linalg.norm(t, axis=1, keepdims=True)
    return t[:_E]


def _router_body(x_ref, wrt_ref, trg_ref, tki_ref, tkw_ref, psum_ref):
    c = pl.program_id(0)
    xb = x_ref[...]
    z3 = lax.dot_general(xb, wrt_ref[...], (((1,), (0,)), ((), ())),
                         precision=lax.Precision.DEFAULT,
                         preferred_element_type=jnp.float32)
    lgT = lax.dot_general(trg_ref[...], z3, (((1,), (1,)), ((), ())),
                          precision=lax.Precision.DEFAULT,
                          preferred_element_type=jnp.float32)
    io8 = lax.broadcasted_iota(jnp.int32, (_E, _TB), 0)
    m1 = jnp.max(lgT, axis=0, keepdims=True)
    a1 = jnp.min(jnp.where(lgT == m1, io8, _E), axis=0, keepdims=True)
    lg2 = jnp.where(io8 == a1, jnp.float32(-1e30), lgT)
    m2 = jnp.max(lg2, axis=0, keepdims=True)
    a2 = jnp.min(jnp.where(lg2 == m2, io8, _E), axis=0, keepdims=True)
    w1 = 1.0 / (1.0 + jnp.exp(m2 - m1))
    tki_ref[0:1, :] = a1
    tki_ref[1:2, :] = a2
    tkw_ref[0:1, :] = w1
    tkw_ref[1:2, :] = 1.0 - w1
    el = jnp.exp(lgT - m1)
    probs = el / jnp.sum(el, axis=0, keepdims=True)
    ones = jnp.ones((_TB, 128), jnp.float32)
    ps = lax.dot_general(probs, ones, (((1,), (0,)), ((), ())),
                         precision=lax.Precision.HIGHEST,
                         preferred_element_type=jnp.float32)

    @pl.when(c == 0)
    def _():
        psum_ref[...] = jnp.zeros_like(psum_ref)

    psum_ref[...] += ps


def _router(xf, wrt, trg):
    return pl.pallas_call(
        _router_body,
        grid=(_N // _TB,),
        in_specs=[
            pl.BlockSpec((_TB, _D), lambda c: (c, 0)),
            pl.BlockSpec((_D, _E), lambda c: (0, 0)),
            pl.BlockSpec((_E, _E), lambda c: (0, 0)),
        ],
        out_specs=[
            pl.BlockSpec((_K, _TB), lambda c: (0, c)),
            pl.BlockSpec((_K, _TB), lambda c: (0, c)),
            pl.BlockSpec((_E, 128), lambda c: (0, 0)),
        ],
        out_shape=[
            jax.ShapeDtypeStruct((_K, _N), jnp.int32),
            jax.ShapeDtypeStruct((_K, _N), jnp.float32),
            jax.ShapeDtypeStruct((_E, 128), jnp.float32),
        ],
    )(xf, wrt, trg)


def _dispatch(tki, xf):
    mesh = plsc.VectorSubcoreMesh(core_axis_name="c", subcore_axis_name="s")

    @functools.partial(
        pl.kernel,
        out_type=[
            jax.ShapeDtypeStruct((_AMAX, _D), jnp.float32),
            jax.ShapeDtypeStruct((_K, _N), jnp.int32),
            jax.ShapeDtypeStruct((_META,), jnp.int32),
        ],
        mesh=mesh,
        scratch_types=[
            pltpu.VMEM((_K * _N,), jnp.int32),
            pltpu.VMEM((4, 16), jnp.int32),
            pltpu.VMEM((4, 16), jnp.int32),
            pltpu.VMEM((_K, _CT), jnp.int32),
            pltpu.VMEM((4, 64), jnp.int32),
            pltpu.VMEM((64, _D), jnp.float32),
            pltpu.VMEM((64, _D), jnp.float32),
            pltpu.VMEM((_META,), jnp.int32),
            pltpu.SemaphoreType.DMA,
            pltpu.SemaphoreType.DMA,
            pltpu.SemaphoreType.DMA,
        ],
        compiler_params=_SC_PARAMS,
    )
    def k(tki_hbm, x_hbm, xs_hbm, dest_hbm, meta_hbm,
          ids_v, sv_v, hst_v, dst_v, idx_v, xb_v, xb2_v, meta_v,
          sem0, sem1, sem2):
        cid = lax.axis_index("c")
        sid = lax.axis_index("s")
        c = sid * 2 + cid
        bufs = [xb_v, xb2_v]
        lds = [pltpu.async_copy(x_hbm.at[pl.ds(c * _CT + s * 64, 64)],
                                bufs[s], sem2) for s in range(2)]
        for kk in range(_K):
            pltpu.sync_copy(tki_hbm.at[kk], ids_v.at[pl.ds(kk * _N, _N)])

        lane = lax.iota(jnp.int32, 16)
        zero16 = jnp.zeros((16,), jnp.int32)

        for r in range(3):
            hst_v[r, :] = zero16
        one16 = jnp.ones((16,), jnp.int32)

        def scan_body(i, _):
            vec = ids_v[pl.ds(i * 16, 16)]
            m0 = jnp.full((16,), i < 8 * c, jnp.bool_)
            m1b = jnp.full((16,), i < 256 + 8 * c, jnp.bool_)
            plsc.addupdate_scatter(hst_v.at[0], [vec], one16)
            plsc.addupdate_scatter(hst_v.at[1], [vec], one16, mask=m0)
            plsc.addupdate_scatter(hst_v.at[2], [vec], one16, mask=m1b)
            return 0

        lax.fori_loop(0, (_K * _N) // 16, scan_body, 0)
        tot = hst_v[0, :]
        p0 = hst_v[1, :]
        p1 = hst_v[2, :]

        pc = jnp.bitwise_and(tot + (_BLK - 1), jnp.int32(-_BLK))
        cum = plsc.cumsum(pc)
        pad_off = cum - pc
        sv_v[0, :] = pad_off + p0
        sv_v[1, :] = pad_off + p1
        sv_v[2, :] = cum

        for kk in range(_K):
            srow = sv_v[kk, :]
            starts = [jnp.full((16,), srow[e], jnp.int32) for e in range(_E)]
            for j in range(_CT // 16):
                vec = ids_v[pl.ds(kk * _N + c * _CT + j * 16, 16)]
                dst = zero16
                for e in range(_E):
                    m = vec == e
                    r = plsc.cumsum(jnp.where(m, 1, 0))
                    dst = dst + jnp.where(m, starts[e] + r - 1, zero16)
                    starts[e] = starts[e] + plsc.all_reduce_population_count(m)
                dst_v[kk, pl.ds(j * 16, 16)] = dst
                idx_v[kk * 2 + j // 4, pl.ds((j % 4) * 16, 16)] = dst
            pltpu.sync_copy(dst_v.at[kk], dest_hbm.at[kk, pl.ds(c * _CT, _CT)])

        @pl.when(c == 0)
        def _():
            cumv = sv_v[2, :]
            pt = cumv[_E - 1]
            nb = lax.shift_right_logical(pt, _BLK.bit_length() - 1)
            s_last = (nb - 1) * _BLK
            eb_last = jnp.int32(0)
            for e in range(_E):
                eb_last = eb_last + jnp.where(cumv[e] <= s_last, 1, 0)
            for j in range(3):
                bi = lane + 16 * j
                s = bi * _BLK
                eb = zero16
                for e in range(_E):
                    eb = eb + jnp.where(jnp.full((16,), cumv[e], jnp.int32) <= s, 1, 0)
                meta_v[pl.ds(16 * j, 16)] = jnp.minimum(eb, jnp.full((16,), eb_last, jnp.int32))
                meta_v[pl.ds(_NB_PAD + 16 * j, 16)] = jnp.minimum(bi, jnp.full((16,), nb - 1, jnp.int32))
            meta_v[pl.ds(96, 16)] = jnp.where(lane == 0, jnp.full((16,), nb, jnp.int32), zero16)
            meta_v[pl.ds(112, 16)] = zero16
            pltpu.sync_copy(meta_v, meta_hbm)

        cps = []
        for sub in range(2):
            lds[sub].wait()
            cps.append(pltpu.async_copy(bufs[sub], xs_hbm.at[idx_v.at[sub]], sem0))
            cps.append(pltpu.async_copy(bufs[sub], xs_hbm.at[idx_v.at[2 + sub]], sem1))
        for cp in cps:
            cp.wait()

    return k(tki, xf)


def _ffn_body(meta_ref, xs_ref, w1_ref, w2_ref, ys_ref):
    b = pl.program_id(0)
    nb = meta_ref[96]

    @pl.when(b < nb)
    def _():
        xg = xs_ref[...]
        fs = _F // _FSPLIT
        acc = jnp.zeros((_BLK, _D), jnp.float32)
        for fi in range(_FSPLIT):
            w1s = w1_ref[0, pl.ds(fi * fs, fs), :]
            h = lax.dot_general(xg, w1s, (((1,), (1,)), ((), ())),
                                precision=lax.Precision.DEFAULT,
                                preferred_element_type=jnp.float32)
            h = 0.5 * h * (1.0 + lax.erf(h * np.float32(1.0 / np.sqrt(2.0))))
            w2s = w2_ref[0, :, pl.ds(fi * fs, fs)]
            acc = acc + lax.dot_general(h, w2s, (((1,), (1,)), ((), ())),
                                        precision=lax.Precision.DEFAULT,
                                        preferred_element_type=jnp.float32)
        ys_ref[...] = acc


def _ffn(meta, xs, W1, W2):
    grid_spec = pltpu.PrefetchScalarGridSpec(
        num_scalar_prefetch=1,
        grid=(_NBMAX,),
        in_specs=[
            pl.BlockSpec((_BLK, _D), lambda b, m: (m[_NB_PAD + b], 0)),
            pl.BlockSpec((1, _F, _D), lambda b, m: (m[b], 0, 0)),
            pl.BlockSpec((1, _D, _F), lambda b, m: (m[b], 0, 0)),
        ],
        out_specs=pl.BlockSpec((_BLK, _D), lambda b, m: (b, 0)),
    )
    return pl.pallas_call(
        _ffn_body,
        grid_spec=grid_spec,
        out_shape=jax.ShapeDtypeStruct((_AMAX, _D), jnp.float32),
    )(meta, xs, W1, W2)


def _combine(ys, dest, tkw):
    mesh = plsc.VectorSubcoreMesh(core_axis_name="c", subcore_axis_name="s")

    @functools.partial(
        pl.kernel,
        out_type=jax.ShapeDtypeStruct((_N, _D), jnp.float32),
        mesh=mesh,
        scratch_types=[
            pltpu.VMEM((_K, _CT), jnp.int32),
            pltpu.VMEM((_K, _CT), jnp.float32),
            pltpu.VMEM((16, _D), jnp.float32),
            pltpu.VMEM((16, _D), jnp.float32),
            pltpu.VMEM((16, _D), jnp.float32),
            pltpu.VMEM((16, _D), jnp.float32),
            pltpu.VMEM((16, _D), jnp.float32),
            pltpu.VMEM((16, _D), jnp.float32),
            pltpu.SemaphoreType.DMA,
            pltpu.SemaphoreType.DMA,
            pltpu.SemaphoreType.DMA,
            pltpu.SemaphoreType.DMA,
        ],
        compiler_params=_SC_PARAMS,
    )
    def k(ys_hbm, dest_hbm, tkw_hbm, out_hbm,
          di_v, w_v, r00, r01, r10, r11, ob0, ob1,
          gs0, gs1, os0, os1):
        cid = lax.axis_index("c")
        sid = lax.axis_index("s")
        c = sid * 2 + cid
        base = c * _CT
        for kk in range(_K):
            pltpu.sync_copy(dest_hbm.at[kk, pl.ds(base, _CT)], di_v.at[kk])
            pltpu.sync_copy(tkw_hbm.at[kk, pl.ds(base, _CT)], w_v.at[kk])

        rbuf = [[r00, r01], [r10, r11]]
        obuf = [ob0, ob1]
        gsem = [gs0, gs1]
        osem = [os0, os1]
        nsub = _CT // 16
        ndeep = 2

        def fire(sub):
            pb = sub % ndeep
            g0 = pltpu.async_copy(ys_hbm.at[di_v.at[0, pl.ds(sub * 16, 16)]],
                                  rbuf[0][pb], gsem[pb])
            g1 = pltpu.async_copy(ys_hbm.at[di_v.at[1, pl.ds(sub * 16, 16)]],
                                  rbuf[1][pb], gsem[pb])
            return g0, g1

        cps = {0: fire(0)}
        ocps = {}
        for sub in range(nsub):
            pb = sub % ndeep
            if sub + 1 < nsub:
                cps[sub + 1] = fire(sub + 1)
            g0, g1 = cps[sub]
            g0.wait()
            g1.wait()
            if sub >= 2:
                ocps[sub - 2].wait()
            w0v = w_v[0, pl.ds(sub * 16, 16)]
            w1v = w_v[1, pl.ds(sub * 16, 16)]
            ob_i = sub % 2
            r0, r1, ob = rbuf[0][pb], rbuf[1][pb], obuf[ob_i]
            for i in range(16):
                w0 = w0v[i]
                w1 = w1v[i]

                @pl.loop(0, _D, step=16)
                def _(d0, i=i, w0=w0, w1=w1, r0=r0, r1=r1, ob=ob):
                    ob[i, pl.ds(d0, 16)] = (w0 * r0[i, pl.ds(d0, 16)]
                                            + w1 * r1[i, pl.ds(d0, 16)])

            ocps[sub] = pltpu.async_copy(ob, out_hbm.at[pl.ds(base + sub * 16, 16)],
                                         osem[ob_i])
        ocps[nsub - 2].wait()
        ocps[nsub - 1].wait()

    return k(ys, dest, tkw)


def kernel(x, Wr, W1, W2):
    xf = x.reshape(_N, _D)
    wrt = jnp.concatenate([Wr.T, jnp.zeros((_D, _E - 3), jnp.float32)], axis=1)
    trg = jnp.asarray(np.pad(_trig_np(), ((0, 0), (0, _E - 3))))
    tki, tkw, psum = _router(xf, wrt, trg)
    xs, dest, meta = _dispatch(tki, xf)
    ys = _ffn(meta, xs, W1, W2)
    out = _combine(ys, dest, tkw)
    tpe = psum[:, 0] / jnp.float32(_N)
    aux = 0.01 * jnp.mean((tpe - jnp.float32(1.0 / _E)) ** 2)
    return out.reshape(_B, _T, _D), aux
